# baseline (speedup 1.0000x reference)
import functools

import jax
import jax.numpy as jnp
from jax import lax
from jax.experimental import pallas as pl
from jax.experimental.pallas import tpu as pltpu
from jax.experimental.pallas import tpu_sc as plsc

_NC = 2
_SPB = 8192
_NBUF = 2


def _sc_diag(x_hbm, pos_hbm, out_hbm, spb, sem_in, sem_out):
    rows = x_hbm.shape[0]
    rows_per_core = rows // _NC
    n_blocks = rows_per_core // _SPB

    cid = lax.axis_index("c")
    sid = lax.axis_index("s")
    c_base = cid * rows_per_core

    def start_in(k, b):
        pltpu.async_copy(
            x_hbm.at[pl.ds(c_base + k * _SPB, _SPB)], spb.at[b], sem_in.at[b])

    def wait_in(k, b):
        pltpu.make_async_copy(
            x_hbm.at[pl.ds(c_base + k * _SPB, _SPB)], spb.at[b],
            sem_in.at[b]).wait()

    def start_out(k, b):
        pltpu.async_copy(
            spb.at[b], out_hbm.at[pl.ds(c_base + k * _SPB, _SPB)],
            sem_out.at[b])

    def wait_out(k, b):
        pltpu.make_async_copy(
            spb.at[b], out_hbm.at[pl.ds(c_base + k * _SPB, _SPB)],
            sem_out.at[b]).wait()

    @pl.when(sid == 0)
    def _():
        start_in(0, 0)

        def body(k, _):
            b = lax.rem(k, _NBUF)
            nb = lax.rem(k + 1, _NBUF)
            wait_in(k, b)

            @pl.when(k >= 1)
            def _():
                wait_out(k - 1, nb)

            @pl.when(k + 1 < n_blocks)
            def _():
                start_in(k + 1, nb)

            start_out(k, b)
            return 0

        lax.fori_loop(0, n_blocks, body, 0)
        wait_out(n_blocks - 1, lax.rem(n_blocks - 1, _NBUF))


def kernel(x, pos_emb):
    B, L, D = x.shape
    x2 = x.reshape(B * L, D)
    run = functools.partial(
        pl.kernel,
        out_type=jax.ShapeDtypeStruct((B * L, D), x.dtype),
        mesh=plsc.VectorSubcoreMesh(core_axis_name="c", subcore_axis_name="s"),
        scratch_types=[
            pltpu.VMEM_SHARED((_NBUF, _SPB, D), jnp.float32),
            pltpu.SemaphoreType.DMA((_NBUF,)),
            pltpu.SemaphoreType.DMA((_NBUF,)),
        ],
    )(_sc_diag)
    out = run(x2, pos_emb)
    return out.reshape(B, L, D)


# speedup vs baseline: 1.1894x; 1.1894x over previous
import functools

import jax
import jax.numpy as jnp
from jax import lax
from jax.experimental import pallas as pl
from jax.experimental.pallas import tpu as pltpu
from jax.experimental.pallas import tpu_sc as plsc

_BLK_B = 128


def _sc_gather(pos_hbm, out_hbm, idx_v, row_v, sem):
    cid = lax.axis_index("c")
    sid = lax.axis_index("s")
    L = out_hbm.shape[0]

    @pl.when(jnp.logical_and(cid == 0, sid == 0))
    def _():
        for i in range(L // 16):
            idx_v[pl.ds(i * 16, 16)] = lax.iota(jnp.int32, 16) + i * 16
        pltpu.async_copy(pos_hbm.at[idx_v], row_v, sem).wait()
        pltpu.sync_copy(row_v, out_hbm)


def _tc_add(x_ref, pos_ref, o_ref):
    o_ref[...] = x_ref[...] + pos_ref[...][None, :, :]


def kernel(x, pos_emb):
    B, L, D = x.shape
    pos = pl.kernel(
        _sc_gather,
        out_type=jax.ShapeDtypeStruct((L, D), pos_emb.dtype),
        mesh=plsc.VectorSubcoreMesh(core_axis_name="c", subcore_axis_name="s"),
        scratch_types=[
            pltpu.VMEM((L,), jnp.int32),
            pltpu.VMEM((L, D), pos_emb.dtype),
            pltpu.SemaphoreType.DMA,
        ],
    )(pos_emb)
    return pl.pallas_call(
        _tc_add,
        grid=(B // _BLK_B,),
        in_specs=[
            pl.BlockSpec((_BLK_B, L, D), lambda i: (i, 0, 0)),
            pl.BlockSpec((L, D), lambda i: (0, 0)),
        ],
        out_specs=pl.BlockSpec((_BLK_B, L, D), lambda i: (i, 0, 0)),
        out_shape=jax.ShapeDtypeStruct((B, L, D), x.dtype),
    )(x, pos)


# speedup vs baseline: 1.1990x; 1.0080x over previous
import functools

import jax
import jax.numpy as jnp
from jax import lax
from jax.experimental import pallas as pl
from jax.experimental.pallas import tpu as pltpu
from jax.experimental.pallas import tpu_sc as plsc

_BLK_B = 128


def _sc_gather(pos_hbm, out_hbm, idx_v, row_v, sem):
    cid = lax.axis_index("c")
    sid = lax.axis_index("s")
    L = out_hbm.shape[0]

    @pl.when(jnp.logical_and(cid == 0, sid == 0))
    def _():
        for i in range(L // 16):
            idx_v[pl.ds(i * 16, 16)] = lax.iota(jnp.int32, 16) + i * 16
        pltpu.async_copy(pos_hbm.at[idx_v], row_v, sem).wait()
        pltpu.sync_copy(row_v, out_hbm)


def _tc_add(x_ref, pos_ref, o_ref):
    o_ref[...] = x_ref[...] + pos_ref[...][None, :, :]


def kernel(x, pos_emb):
    B, L, D = x.shape
    pos = pl.kernel(
        _sc_gather,
        out_type=jax.ShapeDtypeStruct((L, D), pos_emb.dtype),
        mesh=plsc.VectorSubcoreMesh(
            core_axis_name="c", subcore_axis_name="s", num_cores=1),
        scratch_types=[
            pltpu.VMEM((L,), jnp.int32),
            pltpu.VMEM((L, D), pos_emb.dtype),
            pltpu.SemaphoreType.DMA,
        ],
    )(pos_emb)
    return pl.pallas_call(
        _tc_add,
        grid=(B // _BLK_B,),
        in_specs=[
            pl.BlockSpec((_BLK_B, L, D), lambda i: (i, 0, 0)),
            pl.BlockSpec((L, D), lambda i: (0, 0)),
        ],
        out_specs=pl.BlockSpec((_BLK_B, L, D), lambda i: (i, 0, 0)),
        out_shape=jax.ShapeDtypeStruct((B, L, D), x.dtype),
    )(x, pos)
